# baseline (speedup 1.0000x reference)
import functools

import jax
import jax.numpy as jnp
from jax import lax
from jax.experimental import pallas as pl
from jax.experimental.pallas import tpu as pltpu
from jax.experimental.pallas import tpu_sc as plsc

MAXLEN = 200
DIM = 64
_GROWS = MAXLEN * DIM // 128


def _sc_lookup_body(table_hbm, idx_hbm, out_hbm, idx_v, rows_v, sem):
    wid = lax.axis_index("s") * 2 + lax.axis_index("c")

    @pl.when(wid == 0)
    def _():
        pltpu.sync_copy(idx_hbm, idx_v)
        pltpu.async_copy(table_hbm.at[idx_v], rows_v, sem).wait()
        pltpu.sync_copy(rows_v, out_hbm)


def _sc_lookup(table128, idx128):
    mesh = plsc.VectorSubcoreMesh(core_axis_name="c", subcore_axis_name="s")
    k = functools.partial(
        pl.kernel,
        mesh=mesh,
        out_type=jax.ShapeDtypeStruct((_GROWS, 128), jnp.float32),
        scratch_types=[
            pltpu.VMEM((_GROWS,), jnp.int32),
            pltpu.VMEM((_GROWS, 128), jnp.float32),
            pltpu.SemaphoreType.DMA,
        ],
    )(_sc_lookup_body)
    return k(table128, idx128)


def _add_kernel(blk, x_ref, pos_ref, o_ref):
    i = pl.program_id(0)
    pos = pos_ref[pl.ds(i * blk, blk), :]
    o_ref[...] = x_ref[...] + pos[:, :, None]


def kernel(x, pos_emb):
    batch, seq_len, dim = x.shape
    table128 = pos_emb[:seq_len].reshape(_GROWS, 128)
    idx128 = jnp.arange(_GROWS, dtype=jnp.int32)
    pos = _sc_lookup(table128, idx128).reshape(seq_len, dim)
    xt = jnp.transpose(x, (1, 2, 0))
    blk = 4
    grid = (seq_len // blk,)
    out = pl.pallas_call(
        lambda *refs: _add_kernel(blk, *refs),
        grid=grid,
        in_specs=[
            pl.BlockSpec((blk, dim, batch), lambda i: (i, 0, 0)),
            pl.BlockSpec((seq_len, dim), lambda i: (0, 0)),
        ],
        out_specs=pl.BlockSpec((blk, dim, batch), lambda i: (i, 0, 0)),
        out_shape=jax.ShapeDtypeStruct((seq_len, dim, batch), x.dtype),
    )(xt, pos)
    return jnp.transpose(out, (2, 0, 1))


# speedup vs baseline: 1.1902x; 1.1902x over previous
import jax
import jax.numpy as jnp
from jax.experimental import pallas as pl
from jax.experimental.pallas import tpu as pltpu


def _add_kernel(blk, x_ref, post_ref, o_ref, pos_scratch):
    i = pl.program_id(0)

    @pl.when(i == 0)
    def _():
        pos_scratch[...] = jnp.swapaxes(post_ref[...], 0, 1)

    pos = pos_scratch[pl.ds(i * blk, blk), :]
    o_ref[...] = x_ref[...] + pos[:, :, None]


def kernel(x, pos_emb):
    batch, seq_len, dim = x.shape
    xt = jnp.transpose(x, (1, 2, 0))
    post = jnp.transpose(pos_emb[:seq_len])
    blk = 8
    grid = (seq_len // blk,)
    out = pl.pallas_call(
        lambda *refs: _add_kernel(blk, *refs),
        grid=grid,
        in_specs=[
            pl.BlockSpec((blk, dim, batch), lambda i: (i, 0, 0)),
            pl.BlockSpec((dim, seq_len), lambda i: (0, 0)),
        ],
        out_specs=pl.BlockSpec((blk, dim, batch), lambda i: (i, 0, 0)),
        out_shape=jax.ShapeDtypeStruct((seq_len, dim, batch), x.dtype),
        scratch_shapes=[pltpu.VMEM((seq_len, dim), x.dtype)],
    )(xt, post)
    return jnp.transpose(out, (2, 0, 1))


# speedup vs baseline: 1.1933x; 1.0026x over previous
import jax
import jax.numpy as jnp
from jax.experimental import pallas as pl
from jax.experimental.pallas import tpu as pltpu


def _add_kernel(blk, x_ref, post_ref, o_ref, pos_scratch):
    i = pl.program_id(0)

    @pl.when(i == 0)
    def _():
        pos_scratch[...] = jnp.swapaxes(post_ref[...], 0, 1)

    pos = pos_scratch[pl.ds(i * blk, blk), :]
    o_ref[...] = x_ref[...] + pos[:, :, None]


def kernel(x, pos_emb):
    batch, seq_len, dim = x.shape
    xt = jnp.transpose(x, (1, 2, 0))
    post = jnp.transpose(pos_emb[:seq_len])
    blk = 10
    grid = (seq_len // blk,)
    out = pl.pallas_call(
        lambda *refs: _add_kernel(blk, *refs),
        grid=grid,
        in_specs=[
            pl.BlockSpec((blk, dim, batch), lambda i: (i, 0, 0)),
            pl.BlockSpec((dim, seq_len), lambda i: (0, 0)),
        ],
        out_specs=pl.BlockSpec((blk, dim, batch), lambda i: (i, 0, 0)),
        out_shape=jax.ShapeDtypeStruct((seq_len, dim, batch), x.dtype),
        scratch_shapes=[pltpu.VMEM((seq_len, dim), x.dtype)],
    )(xt, post)
    return jnp.transpose(out, (2, 0, 1))


# speedup vs baseline: 1.1948x; 1.0013x over previous
import jax
import jax.numpy as jnp
from jax.experimental import pallas as pl
from jax.experimental.pallas import tpu as pltpu


def _add_kernel(blk, x_ref, post_ref, o_ref, pos_scratch):
    i = pl.program_id(0)

    @pl.when(i == 0)
    def _():
        pos_scratch[...] = jnp.swapaxes(post_ref[...], 0, 1)

    pos = pos_scratch[pl.ds(i * blk, blk), :]
    o_ref[...] = x_ref[...] + pos[:, :, None]


def kernel(x, pos_emb):
    batch, seq_len, dim = x.shape
    xt = jnp.transpose(x, (1, 2, 0))
    post = jnp.transpose(pos_emb[:seq_len])
    blk = next(b for b in (10, 8, 5, 4, 2, 1) if seq_len % b == 0)
    grid = (seq_len // blk,)
    out = pl.pallas_call(
        lambda *refs: _add_kernel(blk, *refs),
        grid=grid,
        in_specs=[
            pl.BlockSpec((blk, dim, batch), lambda i: (i, 0, 0)),
            pl.BlockSpec((dim, seq_len), lambda i: (0, 0)),
        ],
        out_specs=pl.BlockSpec((blk, dim, batch), lambda i: (i, 0, 0)),
        out_shape=jax.ShapeDtypeStruct((seq_len, dim, batch), x.dtype),
        scratch_shapes=[pltpu.VMEM((seq_len, dim), x.dtype)],
    )(xt, post)
    return jnp.transpose(out, (2, 0, 1))
